# Initial kernel scaffold; baseline (speedup 1.0000x reference)
import jax
import jax.numpy as jnp
from jax.experimental import pallas as pl


def kernel(pred_bboxes, pred_classes, true_bboxes, true_labels):
    raise NotImplementedError("write your pallas kernel here")


# speedup vs baseline: 2.5334x; 2.5334x over previous
import functools

import jax
import jax.numpy as jnp
from jax import lax
from jax.experimental import pallas as pl
from jax.experimental.pallas import tpu as pltpu
from jax.experimental.pallas import tpu_sc as plsc

_B, _N, _M, _C = 8, 5000, 50, 80
_IOU_THRESHOLD = 0.5
_NPAD = 5120
_PER_W = _NPAD // 4
_CHUNKS = _PER_W // 16
_MPAD = 64


def _sc_body(pred_hbm, true_hbm, out_hbm, predv, truev, trep, stage):
    cid = lax.axis_index("c")
    sid = lax.axis_index("s")
    wid = sid * 2 + cid
    b = wid // 4
    off = (wid % 4) * _PER_W

    pltpu.sync_copy(pred_hbm.at[b, :, pl.ds(off, _PER_W)], predv)
    pltpu.sync_copy(true_hbm.at[b], truev)

    for m in range(_M):
        reps = []
        for q in range(4):
            idx = jnp.full((16,), m, jnp.int32)
            rowq = jnp.full((16,), q, jnp.int32)
            rep = plsc.load_gather(truev, [rowq, idx])
            trep[q * _M + m, :] = rep
            reps.append(rep)
        trep[4 * _M + m, :] = (reps[2] - reps[0]) * (reps[3] - reps[1])

    def chunk(ci, carry):
        acc, cnt = carry
        o = ci * 16
        px1 = predv[0, pl.ds(o, 16)]
        py1 = predv[1, pl.ds(o, 16)]
        px2 = predv[2, pl.ds(o, 16)]
        py2 = predv[3, pl.ds(o, 16)]
        pa = (px2 - px1) * (py2 - py1)

        best_i = jnp.zeros((16,), jnp.float32)
        best_u = jnp.ones((16,), jnp.float32)
        best_m = jnp.zeros((16,), jnp.int32)
        for m in range(_M):
            tx1 = trep[0 * _M + m, :]
            ty1 = trep[1 * _M + m, :]
            tx2 = trep[2 * _M + m, :]
            ty2 = trep[3 * _M + m, :]
            ta = trep[4 * _M + m, :]
            iw = jnp.maximum(jnp.minimum(px2, tx2) - jnp.maximum(px1, tx1), 0.0)
            ih = jnp.maximum(jnp.minimum(py2, ty2) - jnp.maximum(py1, ty1), 0.0)
            inter = iw * ih
            union = (pa + ta) - inter
            better = inter * best_u > best_i * union
            best_i = jnp.where(better, inter, best_i)
            best_u = jnp.where(better, union, best_u)
            best_m = jnp.where(better, m, best_m)

        mask = best_i > _IOU_THRESHOLD * best_u
        per = jnp.zeros((16,), jnp.float32)
        for c in range(4):
            row = jnp.full((16,), c, jnp.int32)
            mt = plsc.load_gather(truev, [row, best_m])
            p = (px1, py1, px2, py2)[c]
            d = p - mt
            ad = jnp.abs(d)
            per = per + jnp.where(ad < 1.0, 0.5 * d * d, ad - 0.5)
        acc = acc + jnp.where(mask, per, 0.0)
        cnt = cnt + jnp.where(mask, 1.0, 0.0)
        return acc, cnt

    acc, cnt = lax.fori_loop(
        0, _CHUNKS, chunk,
        (jnp.zeros((16,), jnp.float32), jnp.zeros((16,), jnp.float32)))
    stage[0, :] = acc
    stage[1, :] = cnt
    pltpu.sync_copy(stage, out_hbm.at[wid])


_sc_match = pl.kernel(
    _sc_body,
    out_type=jax.ShapeDtypeStruct((32, 2, 16), jnp.float32),
    mesh=plsc.VectorSubcoreMesh(core_axis_name="c", subcore_axis_name="s"),
    scratch_types=[
        pltpu.VMEM((4, _PER_W), jnp.float32),
        pltpu.VMEM((4, _MPAD), jnp.float32),
        pltpu.VMEM((5 * _M, 16), jnp.float32),
        pltpu.VMEM((2, 16), jnp.float32),
    ],
    compiler_params=pltpu.CompilerParams(needs_layout_passes=False),
)


def _tc_body(partials_ref, cls_ref, lab_ref, out_ref):
    s = jnp.sum(partials_ref[:, 0, :])
    cnt = jnp.sum(partials_ref[:, 1, :])
    bbox_loss = s / (4.0 * cnt)

    logits = cls_ref[...]
    mx = jnp.max(logits, axis=-1, keepdims=True)
    lse = jnp.log(jnp.sum(jnp.exp(logits - mx), axis=-1, keepdims=True)) + mx
    onehot = lax.broadcasted_iota(jnp.int32, (_B, 128), 1) == lab_ref[...]
    picked = jnp.sum(jnp.where(onehot, logits, 0.0), axis=-1, keepdims=True) - lse
    cls_loss = -jnp.mean(picked)
    out_ref[...] = jnp.broadcast_to(bbox_loss + cls_loss, (1, 1))


_tc_combine = pl.pallas_call(
    _tc_body,
    out_shape=jax.ShapeDtypeStruct((1, 1), jnp.float32),
)


@functools.partial(jax.jit)
def kernel(pred_bboxes, pred_classes, true_bboxes, true_labels):
    pred_t = jnp.transpose(pred_bboxes, (0, 2, 1))
    pred_t = jnp.pad(pred_t, ((0, 0), (0, 0), (0, _NPAD - _N)))
    true_t = jnp.transpose(true_bboxes, (0, 2, 1))
    true_t = jnp.pad(true_t, ((0, 0), (0, 0), (0, _MPAD - _M)))

    partials = _sc_match(pred_t, true_t)

    cls0 = pred_classes[:, 0, :]
    cls0 = jnp.pad(cls0, ((0, 0), (0, 128 - _C)), constant_values=-1e30)
    lab0 = true_labels[:, 0].astype(jnp.int32).reshape(_B, 1)

    out = _tc_combine(partials, cls0, lab0)
    return out[0, 0]


# speedup vs baseline: 5.3980x; 2.1307x over previous
import functools

import jax
import jax.numpy as jnp
from jax import lax
from jax.experimental import pallas as pl
from jax.experimental.pallas import tpu as pltpu
from jax.experimental.pallas import tpu_sc as plsc

_B, _N, _M, _C = 8, 5000, 50, 80
_IOU_THRESHOLD = 0.5
_NPAD = 5120
_PER_W = _NPAD // 4
_CHUNKS = _PER_W // 16
_MPAD = 64


def _sc_body(pred_hbm, true_hbm, out_hbm, predv, truev, trep, stage):
    cid = lax.axis_index("c")
    sid = lax.axis_index("s")
    wid = sid * 2 + cid
    b = wid // 4
    off = (wid % 4) * _PER_W

    pltpu.sync_copy(pred_hbm.at[b, :, pl.ds(off, _PER_W)], predv)
    pltpu.sync_copy(true_hbm.at[b], truev)

    for m in range(_M):
        reps = []
        for q in range(4):
            idx = jnp.full((16,), m, jnp.int32)
            rowq = jnp.full((16,), q, jnp.int32)
            rep = plsc.load_gather(truev, [rowq, idx])
            trep[q * _M + m, :] = rep
            reps.append(rep)
        trep[4 * _M + m, :] = (reps[2] - reps[0]) * (reps[3] - reps[1])

    _K = 2
    _MH = _M // 2

    def chunk(ci, carry):
        acc, cnt = carry
        P = []
        for k in range(_K):
            o = ci * (16 * _K) + k * 16
            px1 = predv[0, pl.ds(o, 16)]
            py1 = predv[1, pl.ds(o, 16)]
            px2 = predv[2, pl.ds(o, 16)]
            py2 = predv[3, pl.ds(o, 16)]
            pa = (px2 - px1) * (py2 - py1)
            P.append((px1, py1, px2, py2, pa))

        best = [[(jnp.zeros((16,), jnp.float32),
                  jnp.ones((16,), jnp.float32),
                  jnp.zeros((16,), jnp.int32))
                 for _ in range(2)] for _ in range(_K)]
        for s in range(_MH):
            for h in range(2):
                m = s + _MH * h
                tx1 = trep[0 * _M + m, :]
                ty1 = trep[1 * _M + m, :]
                tx2 = trep[2 * _M + m, :]
                ty2 = trep[3 * _M + m, :]
                ta = trep[4 * _M + m, :]
                for k in range(_K):
                    px1, py1, px2, py2, pa = P[k]
                    b_i, b_u, b_m = best[k][h]
                    iw = jnp.maximum(
                        jnp.minimum(px2, tx2) - jnp.maximum(px1, tx1), 0.0)
                    ih = jnp.maximum(
                        jnp.minimum(py2, ty2) - jnp.maximum(py1, ty1), 0.0)
                    inter = iw * ih
                    union = (pa + ta) - inter
                    better = inter * b_u > b_i * union
                    best[k][h] = (jnp.where(better, inter, b_i),
                                  jnp.where(better, union, b_u),
                                  jnp.where(better, m, b_m))

        for k in range(_K):
            px1, py1, px2, py2, pa = P[k]
            (ia, ua, ma), (ib, ub, mb) = best[k]
            upper = ib * ua > ia * ub
            best_i = jnp.where(upper, ib, ia)
            best_u = jnp.where(upper, ub, ua)
            best_m = jnp.where(upper, mb, ma)

            mask = best_i > _IOU_THRESHOLD * best_u
            per = jnp.zeros((16,), jnp.float32)
            for c in range(4):
                row = jnp.full((16,), c, jnp.int32)
                mt = plsc.load_gather(truev, [row, best_m])
                p = (px1, py1, px2, py2)[c]
                d = p - mt
                ad = jnp.abs(d)
                per = per + jnp.where(ad < 1.0, 0.5 * d * d, ad - 0.5)
            acc = acc + jnp.where(mask, per, 0.0)
            cnt = cnt + jnp.where(mask, 1.0, 0.0)
        return acc, cnt

    acc, cnt = lax.fori_loop(
        0, _CHUNKS // _K, chunk,
        (jnp.zeros((16,), jnp.float32), jnp.zeros((16,), jnp.float32)))
    stage[0, :] = acc
    stage[1, :] = cnt
    pltpu.sync_copy(stage, out_hbm.at[wid])


_sc_match = pl.kernel(
    _sc_body,
    out_type=jax.ShapeDtypeStruct((32, 2, 16), jnp.float32),
    mesh=plsc.VectorSubcoreMesh(core_axis_name="c", subcore_axis_name="s"),
    scratch_types=[
        pltpu.VMEM((4, _PER_W), jnp.float32),
        pltpu.VMEM((4, _MPAD), jnp.float32),
        pltpu.VMEM((5 * _M, 16), jnp.float32),
        pltpu.VMEM((2, 16), jnp.float32),
    ],
    compiler_params=pltpu.CompilerParams(needs_layout_passes=False),
)


def _tc_body(partials_ref, cls_ref, lab_ref, out_ref):
    s = jnp.sum(partials_ref[:, 0, :])
    cnt = jnp.sum(partials_ref[:, 1, :])
    bbox_loss = s / (4.0 * cnt)

    logits = cls_ref[...]
    mx = jnp.max(logits, axis=-1, keepdims=True)
    lse = jnp.log(jnp.sum(jnp.exp(logits - mx), axis=-1, keepdims=True)) + mx
    onehot = lax.broadcasted_iota(jnp.int32, (_B, 128), 1) == lab_ref[...]
    picked = jnp.sum(jnp.where(onehot, logits, 0.0), axis=-1, keepdims=True) - lse
    cls_loss = -jnp.mean(picked)
    out_ref[...] = jnp.broadcast_to(bbox_loss + cls_loss, (1, 1))


_tc_combine = pl.pallas_call(
    _tc_body,
    out_shape=jax.ShapeDtypeStruct((1, 1), jnp.float32),
)


@functools.partial(jax.jit)
def kernel(pred_bboxes, pred_classes, true_bboxes, true_labels):
    pred_t = jnp.transpose(pred_bboxes, (0, 2, 1))
    pred_t = jnp.pad(pred_t, ((0, 0), (0, 0), (0, _NPAD - _N)))
    true_t = jnp.transpose(true_bboxes, (0, 2, 1))
    true_t = jnp.pad(true_t, ((0, 0), (0, 0), (0, _MPAD - _M)))

    partials = _sc_match(pred_t, true_t)

    cls0 = pred_classes[:, 0, :]
    cls0 = jnp.pad(cls0, ((0, 0), (0, 128 - _C)), constant_values=-1e30)
    lab0 = true_labels[:, 0].astype(jnp.int32).reshape(_B, 1)

    out = _tc_combine(partials, cls0, lab0)
    return out[0, 0]
